# baseline (speedup 1.0000x reference)
import functools

import jax
import jax.numpy as jnp
from jax import lax
from jax.experimental import pallas as pl
from jax.experimental.pallas import tpu as pltpu
from jax.experimental.pallas import tpu_sc as plsc

_NUM_CLASS = 100000
_D = 512
_B = 1024
_DEPTH_M1 = 8
_SEQ = 77
_NC = 2
_NS = 16
_NW = _NC * _NS
_BPW = _B // _NW

_SC_MESH = plsc.VectorSubcoreMesh(core_axis_name="c", subcore_axis_name="s")


def _worker_base():
    wid = lax.axis_index("s") * _NC + lax.axis_index("c")
    return wid * _BPW


def _sc_gather_cls_body(tab_hbm, idx_hbm, out, idx_v, rows_v, sem):
    base = _worker_base()
    pltpu.sync_copy(idx_hbm.at[pl.ds(base, _BPW)], idx_v)
    pltpu.async_copy(tab_hbm.at[idx_v], rows_v, sem).wait()
    pltpu.sync_copy(rows_v, out.at[pl.ds(base, _BPW)])


_sc_gather_cls = functools.partial(
    pl.kernel,
    out_type=jax.ShapeDtypeStruct((_B, 1, _D), jnp.float32),
    mesh=_SC_MESH,
    scratch_types=[
        pltpu.VMEM((_BPW,), jnp.int32),
        pltpu.VMEM((_BPW, 1, _D), jnp.float32),
        pltpu.SemaphoreType.DMA,
    ],
)(_sc_gather_cls_body)


def _sc_gather_deep_body(tab_hbm, idx_hbm, *rest):
    outs = rest[:_DEPTH_M1]
    idx_v, rows_v, sem = rest[_DEPTH_M1:]
    base = _worker_base()
    for t in range(_DEPTH_M1):
        pltpu.sync_copy(idx_hbm.at[pl.ds(t * _B + base, _BPW)], idx_v)
        pltpu.async_copy(tab_hbm.at[idx_v], rows_v, sem).wait()
        pltpu.sync_copy(rows_v, outs[t].at[pl.ds(base, _BPW)])


_sc_gather_deep = functools.partial(
    pl.kernel,
    out_type=[jax.ShapeDtypeStruct((_B, 1, _D), jnp.float32)
              for _ in range(_DEPTH_M1)],
    mesh=_SC_MESH,
    scratch_types=[
        pltpu.VMEM((_BPW,), jnp.int32),
        pltpu.VMEM((_BPW, 1, _D), jnp.float32),
        pltpu.SemaphoreType.DMA,
    ],
)(_sc_gather_deep_body)


_RPB = 11


def _tc_assemble_body(r_ref, g_ref, o_ref):
    i = pl.program_id(0)
    for r in range(_RPB):
        s = i * _RPB + r

        @pl.when(s == 7)
        def _():
            o_ref[r] = g_ref[:, 0, :]

        @pl.when(s != 7)
        def _():
            o_ref[r] = jnp.broadcast_to(r_ref[r, 0], (_B, _D))


def _tc_assemble(rows, g):
    return pl.pallas_call(
        _tc_assemble_body,
        grid=(_SEQ // _RPB,),
        in_specs=[
            pl.BlockSpec((_RPB, 1, _D), lambda i: (i, 0, 0)),
            pl.BlockSpec((_B, 1, _D), lambda i: (0, 0, 0)),
        ],
        out_specs=pl.BlockSpec((_RPB, _B, _D), lambda i: (i, 0, 0)),
        out_shape=jax.ShapeDtypeStruct((_SEQ, _B, _D), jnp.float32),
    )(rows, g)


def kernel(cls_ctx_per_id, cls_vector, compound_prompts_text,
           compound_per_id_prompts_text, token_prefix, token_suffix, label):
    deep_flat = compound_per_id_prompts_text.reshape(
        _DEPTH_M1 * _NUM_CLASS, 1, _D)
    lbl = label.astype(jnp.int32)
    offs = jnp.arange(_DEPTH_M1, dtype=jnp.int32) * _NUM_CLASS
    idx_deep = (offs[:, None] + lbl[None, :]).reshape(-1)

    g_cls = _sc_gather_cls(cls_ctx_per_id, lbl)
    g_deep = _sc_gather_deep(deep_flat, idx_deep)

    rows = jnp.concatenate(
        [token_prefix[0], cls_vector, cls_vector[:1], token_suffix[0]],
        axis=0).reshape(_SEQ, 1, _D)

    prompts = _tc_assemble(rows, g_cls).transpose(1, 0, 2)

    compound_prompts = tuple(compound_prompts_text[i] for i in range(_DEPTH_M1))
    return (prompts, compound_prompts, tuple(g_deep))


# speedup vs baseline: 1.0080x; 1.0080x over previous
import functools

import jax
import jax.numpy as jnp
from jax import lax
from jax.experimental import pallas as pl
from jax.experimental.pallas import tpu as pltpu
from jax.experimental.pallas import tpu_sc as plsc

_NUM_CLASS = 100000
_D = 512
_B = 1024
_DEPTH_M1 = 8
_SEQ = 77
_NC = 2
_NS = 16
_NW = _NC * _NS
_BPW = _B // _NW

_SC_MESH = plsc.VectorSubcoreMesh(core_axis_name="c", subcore_axis_name="s")


def _worker_base():
    wid = lax.axis_index("s") * _NC + lax.axis_index("c")
    return wid * _BPW


def _sc_gather_cls_body(tab_hbm, idx_hbm, out, idx_v, rows_v, sem):
    base = _worker_base()
    pltpu.sync_copy(idx_hbm.at[pl.ds(base, _BPW)], idx_v)
    pltpu.async_copy(tab_hbm.at[idx_v], rows_v, sem).wait()
    pltpu.sync_copy(rows_v, out.at[pl.ds(base, _BPW)])


_sc_gather_cls = functools.partial(
    pl.kernel,
    out_type=jax.ShapeDtypeStruct((_B, 1, _D), jnp.float32),
    mesh=_SC_MESH,
    scratch_types=[
        pltpu.VMEM((_BPW,), jnp.int32),
        pltpu.VMEM((_BPW, 1, _D), jnp.float32),
        pltpu.SemaphoreType.DMA,
    ],
)(_sc_gather_cls_body)


def _sc_gather_deep_body(tab_hbm, idx_hbm, *rest):
    outs = rest[:_DEPTH_M1]
    idx_v, rows_v, sem = rest[_DEPTH_M1:]
    base = _worker_base()
    for t in range(_DEPTH_M1):
        pltpu.sync_copy(idx_hbm.at[pl.ds(t * _B + base, _BPW)], idx_v)
        pltpu.async_copy(tab_hbm.at[idx_v], rows_v, sem).wait()
        pltpu.sync_copy(rows_v, outs[t].at[pl.ds(base, _BPW)])


_sc_gather_deep = functools.partial(
    pl.kernel,
    out_type=[jax.ShapeDtypeStruct((_B, 1, _D), jnp.float32)
              for _ in range(_DEPTH_M1)],
    mesh=_SC_MESH,
    scratch_types=[
        pltpu.VMEM((_BPW,), jnp.int32),
        pltpu.VMEM((_BPW, 1, _D), jnp.float32),
        pltpu.SemaphoreType.DMA,
    ],
)(_sc_gather_deep_body)


_RPB = 7


def _tc_assemble_body(r_ref, g_ref, o_ref):
    i = pl.program_id(0)
    for r in range(_RPB):
        s = i * _RPB + r

        @pl.when(s == 7)
        def _():
            o_ref[r] = g_ref[:, 0, :]

        @pl.when(s != 7)
        def _():
            o_ref[r] = jnp.broadcast_to(r_ref[r, 0], (_B, _D))


def _tc_assemble(rows, g):
    return pl.pallas_call(
        _tc_assemble_body,
        grid=(_SEQ // _RPB,),
        in_specs=[
            pl.BlockSpec((_RPB, 1, _D), lambda i: (i, 0, 0)),
            pl.BlockSpec((_B, 1, _D), lambda i: (0, 0, 0)),
        ],
        out_specs=pl.BlockSpec((_RPB, _B, _D), lambda i: (i, 0, 0)),
        out_shape=jax.ShapeDtypeStruct((_SEQ, _B, _D), jnp.float32),
    )(rows, g)


def kernel(cls_ctx_per_id, cls_vector, compound_prompts_text,
           compound_per_id_prompts_text, token_prefix, token_suffix, label):
    deep_flat = compound_per_id_prompts_text.reshape(
        _DEPTH_M1 * _NUM_CLASS, 1, _D)
    lbl = label.astype(jnp.int32)
    offs = jnp.arange(_DEPTH_M1, dtype=jnp.int32) * _NUM_CLASS
    idx_deep = (offs[:, None] + lbl[None, :]).reshape(-1)

    g_cls = _sc_gather_cls(cls_ctx_per_id, lbl)
    g_deep = _sc_gather_deep(deep_flat, idx_deep)

    rows = jnp.concatenate(
        [token_prefix[0], cls_vector, cls_vector[:1], token_suffix[0]],
        axis=0).reshape(_SEQ, 1, _D)

    prompts = _tc_assemble(rows, g_cls).transpose(1, 0, 2)

    compound_prompts = tuple(compound_prompts_text[i] for i in range(_DEPTH_M1))
    return (prompts, compound_prompts, tuple(g_deep))
